# baseline (speedup 1.0000x reference)
import functools

import jax
import jax.numpy as jnp
from jax import lax
from jax.experimental import pallas as pl
from jax.experimental.pallas import tpu as pltpu
from jax.experimental.pallas import tpu_sc as plsc

BATCH = 16384
EMBED_DIM = 64
VOCAB = 100000

_NC = 2
_NS = 16
_NW = _NC * _NS
_NBLK = (VOCAB + 127) // 128
_OWN = 25
_NF = 2
_NG = (_OWN + _NF - 1) // _NF
_NBUF = 3
_JMAX = _NBLK - 1
_NCH = 8
_VPC = BATCH // 16 // _NCH
_CAP = _VPC * 16

_mesh = plsc.VectorSubcoreMesh(core_axis_name="c", subcore_axis_name="s")


@functools.partial(
    pl.kernel,
    mesh=_mesh,
    out_type=jax.ShapeDtypeStruct((BATCH, 128), jnp.float32),
    scratch_types=[
        pltpu.VMEM((BATCH,), jnp.int32),
        pltpu.VMEM((BATCH + 64,), jnp.int32),
        pltpu.VMEM((BATCH + 64,), jnp.int32),
        pltpu.VMEM((_NBUF, _NF, 64, 128), jnp.float32),
        pltpu.VMEM((2, 16, 128), jnp.float32),
        pltpu.SemaphoreType.DMA,
        pltpu.SemaphoreType.DMA,
        pltpu.SemaphoreType.DMA,
        pltpu.SemaphoreType.DMA,
        pltpu.SemaphoreType.DMA,
    ],
    compiler_params=pltpu.CompilerParams(needs_layout_passes=False),
)
def _embedding_gather(idx_hbm, table_t_hbm, wide_hbm, idsb, mlist, arena,
                      stage, rowb, sem_s0, sem_s1, sem_s2, sem_w0, sem_w1):
    wid = lax.axis_index("s") * _NC + lax.axis_index("c")
    lo_j = wid * _OWN
    lo_id = lo_j * 128
    hi_id = lo_id + _OWN * 128
    iota16 = lax.iota(jnp.int32, 16)
    sem_s = (sem_s0, sem_s1, sem_s2)
    sem_w = (sem_w0, sem_w1)

    def fetch(g, buf):
        for f in range(_NF):
            j = jnp.minimum(lo_j + (g * _NF + f), _JMAX)
            pltpu.async_copy(
                table_t_hbm.at[:, pl.ds(pl.multiple_of(j * 128, 128), 128)],
                stage.at[buf, f],
                sem_s[buf],
            )

    def drain_stage(buf):
        pltpu.make_async_copy(
            table_t_hbm.at[:, pl.ds(0, 128 * _NF)],
            stage.at[buf],
            sem_s[buf],
        ).wait()

    fetch(0, 0)
    fetch(1, 1)
    pltpu.sync_copy(idx_hbm, idsb)

    def scan_body(v, cnts):
        new = []
        for c in range(_NCH):
            idv = idsb[pl.ds((c * _VPC + v) * 16, 16)]
            posv = (c * _VPC + v) * 16 + iota16
            m = (idv >= lo_id) & (idv < hi_id)
            cum = plsc.cumsum(m.astype(jnp.int32))
            jl = (idv >> 7) - lo_j
            packed = posv | ((idv & 127) << 14) | (jl << 21)
            plsc.store_scatter(
                mlist, [c * _CAP + cnts[c] + cum - 1], packed, mask=m
            )
            new.append(cnts[c] + cum[15])
        return tuple(new)

    cnts = lax.fori_loop(0, _VPC, scan_body, (0,) * _NCH)

    counts = (0,) * _NG

    def make_count(c):
        def count_body(v, counts):
            val = mlist[pl.ds(c * _CAP + v * 16, 16)]
            valid = (v * 16 + iota16) < cnts[c]
            g = (val >> 22) & 15
            return tuple(
                counts[b]
                + plsc.all_reduce_population_count(valid & (g == b))[0]
                for b in range(_NG)
            )
        return count_body

    for c in range(_NCH):
        counts = lax.fori_loop(0, (cnts[c] + 15) // 16, make_count(c), counts)

    bases = []
    acc = 0
    for b in range(_NG):
        bases.append(acc)
        acc = acc + counts[b]

    def make_bucket(c):
        def bucket_body(v, cursors):
            val = mlist[pl.ds(c * _CAP + v * 16, 16)]
            valid = (v * 16 + iota16) < cnts[c]
            g = (val >> 22) & 15
            new = []
            for b in range(_NG):
                m = valid & (g == b)
                cum = plsc.cumsum(m.astype(jnp.int32))
                plsc.store_scatter(arena, [cursors[b] + cum - 1], val, mask=m)
                new.append(cursors[b] + cum[15])
            return tuple(new)
        return bucket_body

    cursors = tuple(bases)
    for c in range(_NCH):
        cursors = lax.fori_loop(
            0, (cnts[c] + 15) // 16, make_bucket(c), cursors
        )

    def make_extract(g):
        base_g = bases[g]
        end_g = base_g + counts[g]

        def extract_body(v, w):
            slot = base_g + v * 16
            val = arena[pl.ds(slot, 16)]
            valid = (slot + iota16) < end_g
            posv = jnp.where(valid, val & 16383, -1)
            lv = (val >> 14) & 127
            bv = (val >> 21) & 1

            def wave(p):
                @pl.when(w >= 2)
                def _():
                    pltpu.make_async_copy(
                        wide_hbm.at[pl.ds(0, 16)], rowb.at[p], sem_w[p]
                    ).wait()

                for i in range(16):
                    bi = jnp.full((16,), bv[i], jnp.int32)
                    li = jnp.full((16,), lv[i], jnp.int32)
                    for c0 in range(0, 64, 16):
                        rowb[p, i, pl.ds(c0, 16)] = plsc.load_gather(
                            stage.at[g % _NBUF], [bi, c0 + iota16, li]
                        )
                pltpu.async_copy(
                    rowb.at[p],
                    wide_hbm.at[plsc.Indices(posv, ignored_value=-1)],
                    sem_w[p],
                )

            @pl.when(w % 2 == 0)
            def _():
                wave(0)

            @pl.when(w % 2 == 1)
            def _():
                wave(1)

            return w + 1

        return extract_body

    w = 0
    for g in range(_NG):
        if g + 2 < _NG:
            fetch(g + 2, (g + 2) % _NBUF)
        drain_stage(g % _NBUF)
        nvg = (counts[g] + 15) // 16
        w = lax.fori_loop(0, nvg, make_extract(g), w)

    for q in range(2):
        @pl.when((w >= 2) | ((w >= 1) & ((w - 1) % 2 == q)))
        def _(q=q):
            pltpu.make_async_copy(
                wide_hbm.at[pl.ds(0, 16)], rowb.at[q], sem_w[q]
            ).wait()


def kernel(domain_ids, table):
    wide = _embedding_gather(domain_ids.astype(jnp.int32), table.T)
    return wide[:, :EMBED_DIM]


# speedup vs baseline: 1.0970x; 1.0970x over previous
import functools

import jax
import jax.numpy as jnp
from jax import lax
from jax.experimental import pallas as pl
from jax.experimental.pallas import tpu as pltpu
from jax.experimental.pallas import tpu_sc as plsc

BATCH = 16384
EMBED_DIM = 64
VOCAB = 100000

_NC = 2
_NS = 16
_NW = _NC * _NS
_NBLK = (VOCAB + 127) // 128
_OWN = 25
_NF = 4
_NG = (_OWN + _NF - 1) // _NF
_NBUF = 2
_JMAX = _NBLK - 1
_NCH = 8
_VPC = BATCH // 16 // _NCH
_CAP = _VPC * 16

_mesh = plsc.VectorSubcoreMesh(core_axis_name="c", subcore_axis_name="s")


@functools.partial(
    pl.kernel,
    mesh=_mesh,
    out_type=jax.ShapeDtypeStruct((BATCH, 128), jnp.float32),
    scratch_types=[
        pltpu.VMEM((BATCH,), jnp.int32),
        pltpu.VMEM((BATCH + 64,), jnp.int32),
        pltpu.VMEM((BATCH + 64,), jnp.int32),
        pltpu.VMEM((_NBUF, _NF, 64, 128), jnp.float32),
        pltpu.VMEM((2, 16, 128), jnp.float32),
        pltpu.SemaphoreType.DMA,
        pltpu.SemaphoreType.DMA,
        pltpu.SemaphoreType.DMA,
        pltpu.SemaphoreType.DMA,
    ],
    compiler_params=pltpu.CompilerParams(needs_layout_passes=False),
)
def _embedding_gather(idx_hbm, table_t_hbm, wide_hbm, idsb, mlist, arena,
                      stage, rowb, sem_s0, sem_s1, sem_w0, sem_w1):
    wid = lax.axis_index("s") * _NC + lax.axis_index("c")
    lo_j = wid * _OWN
    lo_id = lo_j * 128
    hi_id = lo_id + _OWN * 128
    iota16 = lax.iota(jnp.int32, 16)
    sem_s = (sem_s0, sem_s1)
    sem_w = (sem_w0, sem_w1)

    def fetch(g, buf):
        for f in range(_NF):
            j = jnp.minimum(lo_j + (g * _NF + f), _JMAX)
            pltpu.async_copy(
                table_t_hbm.at[:, pl.ds(pl.multiple_of(j * 128, 128), 128)],
                stage.at[buf, f],
                sem_s[buf],
            )

    def drain_stage(buf):
        pltpu.make_async_copy(
            table_t_hbm.at[:, pl.ds(0, 128 * _NF)],
            stage.at[buf],
            sem_s[buf],
        ).wait()

    fetch(0, 0)
    pltpu.sync_copy(idx_hbm, idsb)

    def scan_body(v, cnts):
        new = []
        for c in range(_NCH):
            idv = idsb[pl.ds((c * _VPC + v) * 16, 16)]
            posv = (c * _VPC + v) * 16 + iota16
            m = (idv >= lo_id) & (idv < hi_id)
            cum = plsc.cumsum(m.astype(jnp.int32))
            jl = (idv >> 7) - lo_j
            packed = posv | ((idv & 127) << 14) | (jl << 21)
            plsc.store_scatter(
                mlist, [c * _CAP + cnts[c] + cum - 1], packed, mask=m
            )
            new.append(cnts[c] + cum[15])
        return tuple(new)

    cnts = lax.fori_loop(0, _VPC, scan_body, (0,) * _NCH)

    counts = (0,) * _NG

    def make_count(c):
        def count_body(v, counts):
            val = mlist[pl.ds(c * _CAP + v * 16, 16)]
            valid = (v * 16 + iota16) < cnts[c]
            g = (val >> 23) & 7
            return tuple(
                counts[b]
                + plsc.all_reduce_population_count(valid & (g == b))[0]
                for b in range(_NG)
            )
        return count_body

    for c in range(_NCH):
        counts = lax.fori_loop(0, (cnts[c] + 15) // 16, make_count(c), counts)

    bases = []
    acc = 0
    for b in range(_NG):
        bases.append(acc)
        acc = acc + counts[b]

    def make_bucket(c):
        def bucket_body(v, cursors):
            val = mlist[pl.ds(c * _CAP + v * 16, 16)]
            valid = (v * 16 + iota16) < cnts[c]
            g = (val >> 23) & 7
            new = []
            for b in range(_NG):
                m = valid & (g == b)
                cum = plsc.cumsum(m.astype(jnp.int32))
                plsc.store_scatter(arena, [cursors[b] + cum - 1], val, mask=m)
                new.append(cursors[b] + cum[15])
            return tuple(new)
        return bucket_body

    cursors = tuple(bases)
    for c in range(_NCH):
        cursors = lax.fori_loop(
            0, (cnts[c] + 15) // 16, make_bucket(c), cursors
        )

    def make_extract(g):
        base_g = bases[g]
        end_g = base_g + counts[g]

        def extract_body(v, w):
            slot = base_g + v * 16
            val = arena[pl.ds(slot, 16)]
            valid = (slot + iota16) < end_g
            posv = jnp.where(valid, val & 16383, -1)
            lv = (val >> 14) & 127
            bv = (val >> 21) & 3

            def wave(p):
                @pl.when(w >= 2)
                def _():
                    pltpu.make_async_copy(
                        wide_hbm.at[pl.ds(0, 16)], rowb.at[p], sem_w[p]
                    ).wait()

                for c0 in range(EMBED_DIM):
                    cvec = jnp.full((16,), c0, jnp.int32)
                    vals = plsc.load_gather(
                        stage.at[g % _NBUF], [bv, cvec, lv]
                    )
                    plsc.store_scatter(rowb.at[p], [iota16, cvec], vals)
                pltpu.async_copy(
                    rowb.at[p],
                    wide_hbm.at[plsc.Indices(posv, ignored_value=-1)],
                    sem_w[p],
                )

            @pl.when(w % 2 == 0)
            def _():
                wave(0)

            @pl.when(w % 2 == 1)
            def _():
                wave(1)

            return w + 1

        return extract_body

    w = 0
    for g in range(_NG):
        if g + 1 < _NG:
            fetch(g + 1, (g + 1) % _NBUF)
        drain_stage(g % _NBUF)
        nvg = (counts[g] + 15) // 16
        w = lax.fori_loop(0, nvg, make_extract(g), w)

    for q in range(2):
        @pl.when((w >= 2) | ((w >= 1) & ((w - 1) % 2 == q)))
        def _(q=q):
            pltpu.make_async_copy(
                wide_hbm.at[pl.ds(0, 16)], rowb.at[q], sem_w[q]
            ).wait()


def kernel(domain_ids, table):
    wide = _embedding_gather(domain_ids.astype(jnp.int32), table.T)
    return wide[:, :EMBED_DIM]
